# Initial kernel scaffold; baseline (speedup 1.0000x reference)
import jax
import jax.numpy as jnp
from jax.experimental import pallas as pl


def kernel(x, edge_index, batch, code_x, W1, b1, W2, b2, Wc, bc, Wt, bt, Wf, bf):
    raise NotImplementedError("write your pallas kernel here")


# speedup vs baseline: 9.9358x; 9.9358x over previous
import functools

import jax
import jax.numpy as jnp
from jax import lax
from jax.experimental import pallas as pl
from jax.experimental.pallas import tpu as pltpu
from jax.experimental.pallas import tpu_sc as plsc

N = 10000
E = 320000
B = 256
D = 128
CODE = 256
FINAL = 128

NPAD = 10240
BR = 512
NB = NPAD // BR

NC = 2
NS = 16
CHUNK = 128
NCHUNK = 79
EPAD = NC * NS * NCHUNK * CHUNK
RPT = NPAD // NS
DEGW = 16

_HIGH = jax.lax.Precision.HIGHEST


def _mesh():
    return plsc.VectorSubcoreMesh(core_axis_name="c", subcore_axis_name="s")


@functools.cache
def _make_deg_kernel():
    return functools.partial(
        pl.kernel,
        mesh=_mesh(),
        out_type=jax.ShapeDtypeStruct((NC, NPAD, D), jnp.float32),
        scratch_types=[
            pltpu.VMEM((CHUNK,), jnp.int32),
            pltpu.VMEM((CHUNK, D), jnp.float32),
            pltpu.VMEM((CHUNK, D), jnp.float32),
            pltpu.VMEM_SHARED((NPAD, D), jnp.float32),
        ],
    )(_deg_body)


def _deg_body(dst_hbm, deg_out, idx_v, ones_v, zero_v, acc_s):
    cid = lax.axis_index("c")
    sid = lax.axis_index("s")

    def fill(i, _):
        for j in range(D // 16):
            ones_v[i, pl.ds(j * 16, 16)] = jnp.full((16,), 1.0, jnp.float32)
            zero_v[i, pl.ds(j * 16, 16)] = jnp.zeros((16,), jnp.float32)
        return 0

    lax.fori_loop(0, CHUNK, fill, 0)
    for r in range(RPT // CHUNK):
        pltpu.sync_copy(zero_v, acc_s.at[pl.ds(sid * RPT + r * CHUNK, CHUNK)])
    plsc.subcore_barrier()

    def body(ci, _):
        pltpu.sync_copy(dst_hbm.at[cid, sid, ci], idx_v)
        pltpu.sync_copy(ones_v, acc_s.at[idx_v], add=True)
        return 0

    lax.fori_loop(0, NCHUNK, body, 0)
    plsc.subcore_barrier()
    pltpu.sync_copy(acc_s.at[pl.ds(sid * RPT, RPT)],
                    deg_out.at[cid, pl.ds(sid * RPT, RPT)])


@functools.cache
def _make_agg_kernel():
    return functools.partial(
        pl.kernel,
        mesh=_mesh(),
        out_type=jax.ShapeDtypeStruct((NC, NPAD, D), jnp.float32),
        scratch_types=[
            pltpu.VMEM((CHUNK,), jnp.int32),
            pltpu.VMEM((CHUNK,), jnp.int32),
            pltpu.VMEM((CHUNK, D), jnp.float32),
            pltpu.VMEM_SHARED((NPAD, D), jnp.float32),
            pltpu.SemaphoreType.DMA,
        ],
    )(_agg_body)


def _agg_body(g_hbm, src_hbm, dst_hbm, out_hbm, src_v, dst_v, rows_v, acc_s, sem):
    cid = lax.axis_index("c")
    sid = lax.axis_index("s")

    def zfill(i, _):
        for j in range(D // 16):
            rows_v[i, pl.ds(j * 16, 16)] = jnp.zeros((16,), jnp.float32)
        return 0

    lax.fori_loop(0, CHUNK, zfill, 0)
    for r in range(RPT // CHUNK):
        pltpu.sync_copy(rows_v, acc_s.at[pl.ds(sid * RPT + r * CHUNK, CHUNK)])
    plsc.subcore_barrier()

    def body(ci, _):
        pltpu.sync_copy(src_hbm.at[cid, sid, ci], src_v)
        pltpu.sync_copy(dst_hbm.at[cid, sid, ci], dst_v)
        pltpu.async_copy(g_hbm.at[src_v], rows_v, sem).wait()
        pltpu.sync_copy(rows_v, acc_s.at[dst_v], add=True)
        return 0

    lax.fori_loop(0, NCHUNK, body, 0)
    plsc.subcore_barrier()
    pltpu.sync_copy(acc_s.at[pl.ds(sid * RPT, RPT)],
                    out_hbm.at[cid, pl.ds(sid * RPT, RPT)])


def _tc1_body(x_ref, w_ref, deg_ref, g_ref, dinv_ref):
    d = deg_ref[0] + deg_ref[1]
    dtot = d[:, 0:1] + 1.0
    dinv = lax.rsqrt(jnp.maximum(dtot, 1.0))
    y = jnp.dot(x_ref[...], w_ref[...],
                preferred_element_type=jnp.float32, precision=_HIGH)
    g_ref[...] = y * dinv
    dinv_ref[...] = jnp.broadcast_to(dinv, (BR, DEGW))


def _tc1(x_pad, W1, degp):
    return pl.pallas_call(
        _tc1_body,
        grid=(NB,),
        in_specs=[
            pl.BlockSpec((BR, D), lambda k: (k, 0)),
            pl.BlockSpec((D, D), lambda k: (0, 0)),
            pl.BlockSpec((NC, BR, D), lambda k: (0, k, 0)),
        ],
        out_specs=[
            pl.BlockSpec((BR, D), lambda k: (k, 0)),
            pl.BlockSpec((BR, DEGW), lambda k: (k, 0)),
        ],
        out_shape=[
            jax.ShapeDtypeStruct((NPAD, D), jnp.float32),
            jax.ShapeDtypeStruct((NPAD, DEGW), jnp.float32),
        ],
    )(x_pad, W1, degp)


def _tc2_body(g_ref, p_ref, dinv_ref, b_ref, w_ref, o_ref):
    dinv = dinv_ref[:, 0:1]
    s = p_ref[0] + p_ref[1] + g_ref[...]
    h = jnp.maximum(s * dinv + b_ref[...], 0.0)
    y = jnp.dot(h, w_ref[...],
                preferred_element_type=jnp.float32, precision=_HIGH)
    o_ref[...] = y * dinv


def _tc2(g1, p1, dinv16, b1, W2):
    return pl.pallas_call(
        _tc2_body,
        grid=(NB,),
        in_specs=[
            pl.BlockSpec((BR, D), lambda k: (k, 0)),
            pl.BlockSpec((NC, BR, D), lambda k: (0, k, 0)),
            pl.BlockSpec((BR, DEGW), lambda k: (k, 0)),
            pl.BlockSpec((1, D), lambda k: (0, 0)),
            pl.BlockSpec((D, D), lambda k: (0, 0)),
        ],
        out_specs=pl.BlockSpec((BR, D), lambda k: (k, 0)),
        out_shape=jax.ShapeDtypeStruct((NPAD, D), jnp.float32),
    )(g1, p1, dinv16, b1, W2)


def _log_softmax(z):
    m = jnp.max(z, axis=-1, keepdims=True)
    zs = z - m
    return zs - jnp.log(jnp.sum(jnp.exp(zs), axis=-1, keepdims=True))


def _tc3_body(g_ref, p_ref, dinv_ref, b2_ref, batch_ref, cx_ref,
              wc_ref, bc_ref, wt_ref, bt_ref, wfc_ref, wft_ref, bf_ref,
              o1_ref, o2_ref, o3_ref, sums, counts):
    k = pl.program_id(0)

    @pl.when(k == 0)
    def _init():
        sums[...] = jnp.zeros_like(sums)
        counts[...] = jnp.zeros_like(counts)

    dinv = dinv_ref[:, 0:1]
    h2 = (p_ref[0] + p_ref[1] + g_ref[...]) * dinv + b2_ref[...]
    bvec = batch_ref[0]
    seg = lax.broadcasted_iota(jnp.int32, (B, BR), 0)
    onehot = (bvec == seg).astype(jnp.float32)
    sums[...] += jnp.dot(onehot, h2,
                         preferred_element_type=jnp.float32, precision=_HIGH)
    counts[...] += jnp.sum(onehot, axis=1, keepdims=True)

    @pl.when(k == NB - 1)
    def _final():
        cnt = jnp.maximum(counts[:, 0:1], 1.0)
        trans = sums[...] / cnt
        code = cx_ref[...]
        z1 = jnp.dot(code, wc_ref[...],
                     preferred_element_type=jnp.float32, precision=_HIGH) + bc_ref[...]
        o1_ref[...] = _log_softmax(z1)
        z2 = jnp.dot(trans, wt_ref[...],
                     preferred_element_type=jnp.float32, precision=_HIGH) + bt_ref[...]
        o2_ref[...] = _log_softmax(z2)
        z3 = (jnp.dot(code, wfc_ref[...],
                      preferred_element_type=jnp.float32, precision=_HIGH)
              + jnp.dot(trans, wft_ref[...],
                        preferred_element_type=jnp.float32, precision=_HIGH)
              + bf_ref[...])
        o3_ref[...] = _log_softmax(z3)


def _tc3(g2, p2, dinv16, b2, batch3, code_x, Wc, bc, Wt, bt, Wfc, Wft, bf):
    return pl.pallas_call(
        _tc3_body,
        grid=(NB,),
        in_specs=[
            pl.BlockSpec((BR, D), lambda k: (k, 0)),
            pl.BlockSpec((NC, BR, D), lambda k: (0, k, 0)),
            pl.BlockSpec((BR, DEGW), lambda k: (k, 0)),
            pl.BlockSpec((1, D), lambda k: (0, 0)),
            pl.BlockSpec((1, 1, BR), lambda k: (k, 0, 0)),
            pl.BlockSpec((B, CODE), lambda k: (0, 0)),
            pl.BlockSpec((CODE, FINAL), lambda k: (0, 0)),
            pl.BlockSpec((1, FINAL), lambda k: (0, 0)),
            pl.BlockSpec((D, FINAL), lambda k: (0, 0)),
            pl.BlockSpec((1, FINAL), lambda k: (0, 0)),
            pl.BlockSpec((CODE, FINAL), lambda k: (0, 0)),
            pl.BlockSpec((D, FINAL), lambda k: (0, 0)),
            pl.BlockSpec((1, FINAL), lambda k: (0, 0)),
        ],
        out_specs=[
            pl.BlockSpec((B, FINAL), lambda k: (0, 0)),
            pl.BlockSpec((B, FINAL), lambda k: (0, 0)),
            pl.BlockSpec((B, FINAL), lambda k: (0, 0)),
        ],
        out_shape=[
            jax.ShapeDtypeStruct((B, FINAL), jnp.float32),
            jax.ShapeDtypeStruct((B, FINAL), jnp.float32),
            jax.ShapeDtypeStruct((B, FINAL), jnp.float32),
        ],
        scratch_shapes=[
            pltpu.VMEM((B, FINAL), jnp.float32),
            pltpu.VMEM((B, FINAL), jnp.float32),
        ],
    )(g2, p2, dinv16, b2, batch3, code_x, Wc, bc, Wt, bt, Wfc, Wft, bf)


def kernel(x, edge_index, batch, code_x, W1, b1, W2, b2, Wc, bc, Wt, bt, Wf, bf):
    x_pad = jnp.pad(x, ((0, NPAD - N), (0, 0)))
    src = edge_index[0].astype(jnp.int32)
    dst = edge_index[1].astype(jnp.int32)
    fill = jnp.full((EPAD - E,), NPAD - 1, jnp.int32)
    src3 = jnp.concatenate([src, fill]).reshape(NC, NS, NCHUNK, CHUNK)
    dst3 = jnp.concatenate([dst, fill]).reshape(NC, NS, NCHUNK, CHUNK)
    batch3 = jnp.concatenate(
        [batch.astype(jnp.int32), jnp.full((NPAD - N,), 1 << 20, jnp.int32)]
    ).reshape(NB, 1, BR)

    b1r = b1.reshape(1, D)
    b2r = b2.reshape(1, D)
    bcr = bc.reshape(1, FINAL)
    btr = bt.reshape(1, FINAL)
    bfr = bf.reshape(1, FINAL)
    Wfc = Wf[:CODE]
    Wft = Wf[CODE:]

    deg_kernel = _make_deg_kernel()
    agg_kernel = _make_agg_kernel()
    degp = deg_kernel(dst3)
    g1, dinv16 = _tc1(x_pad, W1, degp)
    p1 = agg_kernel(g1, src3, dst3)
    g2 = _tc2(g1, p1, dinv16, b1r, W2)
    p2 = agg_kernel(g2, src3, dst3)
    code_prob, trans_prob, final_prob = _tc3(
        g2, p2, dinv16, b2r, batch3, code_x, Wc, bcr, Wt, btr, Wfc, Wft, bfr)
    return (code_prob, trans_prob, final_prob)
